# baseline (speedup 1.0000x reference)
import functools

import jax
import jax.numpy as jnp
from jax import lax
from jax.experimental import pallas as pl
from jax.experimental.pallas import tpu as pltpu
from jax.experimental.pallas import tpu_sc as plsc

EMB = 32
NUM_CORES = 2
NUM_SUBCORES = 16
NUM_WORKERS = NUM_CORES * NUM_SUBCORES
CHUNK = 800
DEPTH = 4


def _build(n):
    per_w = n // NUM_WORKERS
    n_chunks = per_w // CHUNK
    mesh = plsc.VectorSubcoreMesh(
        core_axis_name="c",
        subcore_axis_name="s",
        num_cores=NUM_CORES,
        num_subcores=NUM_SUBCORES,
    )

    @functools.partial(
        pl.kernel,
        out_type=jax.ShapeDtypeStruct((n, EMB), jnp.float32),
        mesh=mesh,
        scratch_types=[
            pltpu.VMEM((DEPTH, CHUNK), jnp.int32),
            pltpu.VMEM((DEPTH, CHUNK, EMB), jnp.float32),
        ]
        + [pltpu.SemaphoreType.DMA] * (3 * DEPTH),
        compiler_params=pltpu.CompilerParams(use_tc_tiling_on_sc=False),
    )
    def emb_kernel(idx_hbm, table_hbm, out_hbm, idx_v, rows_v, *sems):
        wid = lax.axis_index("s") * NUM_CORES + lax.axis_index("c")
        base = wid * per_w
        sem_i = sems[0:DEPTH]
        sem_g = sems[DEPTH : 2 * DEPTH]
        sem_w = sems[2 * DEPTH : 3 * DEPTH]

        idxl = [None] * DEPTH
        gathers = [None] * DEPTH
        writes = [None] * DEPTH
        LAG = DEPTH - 2

        idxl[0] = pltpu.async_copy(
            idx_hbm.at[pl.ds(base, CHUNK)], idx_v.at[0], sem_i[0]
        )
        for i in range(n_chunks):
            b = i % DEPTH
            if i + 1 < n_chunks:
                nb = (i + 1) % DEPTH
                if writes[nb] is not None:
                    writes[nb].wait()
                    writes[nb] = None
                idxl[nb] = pltpu.async_copy(
                    idx_hbm.at[pl.ds(base + (i + 1) * CHUNK, CHUNK)],
                    idx_v.at[nb],
                    sem_i[nb],
                )
            idxl[b].wait()
            gathers[b] = pltpu.async_copy(
                table_hbm.at[idx_v.at[b]], rows_v.at[b], sem_g[b]
            )
            j = i - LAG
            if j >= 0:
                bj = j % DEPTH
                gathers[bj].wait()
                writes[bj] = pltpu.async_copy(
                    rows_v.at[bj],
                    out_hbm.at[pl.ds(base + j * CHUNK, CHUNK)],
                    sem_w[bj],
                )
        for j in range(max(0, n_chunks - LAG), n_chunks):
            bj = j % DEPTH
            gathers[bj].wait()
            writes[bj] = pltpu.async_copy(
                rows_v.at[bj],
                out_hbm.at[pl.ds(base + j * CHUNK, CHUNK)],
                sem_w[bj],
            )
        for w in writes:
            if w is not None:
                w.wait()

    return emb_kernel


SPLIT = 2


def kernel(input, table):
    B, L = input.shape
    n = B * L
    idx = input.reshape(n).astype(jnp.int32)
    h = n // SPLIT
    f = _build(h)
    parts = [f(idx[k * h : (k + 1) * h], table) for k in range(SPLIT)]
    return jnp.concatenate(parts, axis=0).reshape(B, L, EMB)


# speedup vs baseline: 1.3956x; 1.3956x over previous
import functools

import jax
import jax.numpy as jnp
from jax import lax
from jax.experimental import pallas as pl
from jax.experimental.pallas import tpu as pltpu
from jax.experimental.pallas import tpu_sc as plsc

EMB = 32
NUM_CORES = 2
NUM_SUBCORES = 16
NUM_WORKERS = NUM_CORES * NUM_SUBCORES
CHUNK = 1280
DEPTH = 3


def _build(n):
    per_w = n // NUM_WORKERS
    n_chunks = per_w // CHUNK
    mesh = plsc.VectorSubcoreMesh(
        core_axis_name="c",
        subcore_axis_name="s",
        num_cores=NUM_CORES,
        num_subcores=NUM_SUBCORES,
    )

    @functools.partial(
        pl.kernel,
        out_type=jax.ShapeDtypeStruct((n, EMB), jnp.float32),
        mesh=mesh,
        scratch_types=[
            pltpu.VMEM((DEPTH, CHUNK), jnp.int32),
            pltpu.VMEM((DEPTH, CHUNK, EMB), jnp.float32),
        ]
        + [pltpu.SemaphoreType.DMA] * (3 * DEPTH),
        compiler_params=pltpu.CompilerParams(use_tc_tiling_on_sc=False),
    )
    def emb_kernel(idx_hbm, table_hbm, out_hbm, idx_v, rows_v, *sems):
        wid = lax.axis_index("s") * NUM_CORES + lax.axis_index("c")
        base = wid * per_w
        sem_i = sems[0:DEPTH]
        sem_g = sems[DEPTH : 2 * DEPTH]
        sem_w = sems[2 * DEPTH : 3 * DEPTH]

        idxl = [None] * DEPTH
        gathers = [None] * DEPTH
        writes = [None] * DEPTH
        LAG = DEPTH - 2

        idxl[0] = pltpu.async_copy(
            idx_hbm.at[pl.ds(base, CHUNK)], idx_v.at[0], sem_i[0]
        )
        for i in range(n_chunks):
            b = i % DEPTH
            if i + 1 < n_chunks:
                nb = (i + 1) % DEPTH
                if writes[nb] is not None:
                    writes[nb].wait()
                    writes[nb] = None
                idxl[nb] = pltpu.async_copy(
                    idx_hbm.at[pl.ds(base + (i + 1) * CHUNK, CHUNK)],
                    idx_v.at[nb],
                    sem_i[nb],
                )
            idxl[b].wait()
            gathers[b] = pltpu.async_copy(
                table_hbm.at[idx_v.at[b]], rows_v.at[b], sem_g[b]
            )
            j = i - LAG
            if j >= 0:
                bj = j % DEPTH
                gathers[bj].wait()
                writes[bj] = pltpu.async_copy(
                    rows_v.at[bj],
                    out_hbm.at[pl.ds(base + j * CHUNK, CHUNK)],
                    sem_w[bj],
                )
        for j in range(max(0, n_chunks - LAG), n_chunks):
            bj = j % DEPTH
            gathers[bj].wait()
            writes[bj] = pltpu.async_copy(
                rows_v.at[bj],
                out_hbm.at[pl.ds(base + j * CHUNK, CHUNK)],
                sem_w[bj],
            )
        for w in writes:
            if w is not None:
                w.wait()

    return emb_kernel


def kernel(input, table):
    B, L = input.shape
    n = B * L
    idx = input.reshape(n).astype(jnp.int32)
    out = _build(n)(idx, table)
    return out.reshape(B, L, EMB)
